# baseline (speedup 1.0000x reference)
import jax
import jax.numpy as jnp
from jax.experimental import pallas as pl

BATCH = 4
SEQ_LEN = 8192
D_MODEL = 768
BS = 512


def _add_body(x_ref, pos_ref, out_ref):
    out_ref[...] = x_ref[...] + pos_ref[...][None]


def kernel(x, pos_table):
    grid = (SEQ_LEN // BS, BATCH)
    return pl.pallas_call(
        _add_body,
        grid=grid,
        in_specs=[
            pl.BlockSpec((1, BS, D_MODEL), lambda s, b: (b, s, 0)),
            pl.BlockSpec((BS, D_MODEL), lambda s, b: (s, 0)),
        ],
        out_specs=pl.BlockSpec((1, BS, D_MODEL), lambda s, b: (b, s, 0)),
        out_shape=jax.ShapeDtypeStruct((BATCH, SEQ_LEN, D_MODEL), jnp.float32),
    )(x, pos_table)


# speedup vs baseline: 1.2386x; 1.2386x over previous
import jax
import jax.numpy as jnp
from jax.experimental import pallas as pl

BATCH = 4
SEQ_LEN = 8192
D_MODEL = 768
BS = 2048


def _add_body(x_ref, pos_ref, out_ref):
    out_ref[...] = x_ref[...] + pos_ref[...][None]


def kernel(x, pos_table):
    grid = (SEQ_LEN // BS, BATCH)
    return pl.pallas_call(
        _add_body,
        grid=grid,
        in_specs=[
            pl.BlockSpec((1, BS, D_MODEL), lambda s, b: (b, s, 0)),
            pl.BlockSpec((BS, D_MODEL), lambda s, b: (s, 0)),
        ],
        out_specs=pl.BlockSpec((1, BS, D_MODEL), lambda s, b: (b, s, 0)),
        out_shape=jax.ShapeDtypeStruct((BATCH, SEQ_LEN, D_MODEL), jnp.float32),
    )(x, pos_table)


# speedup vs baseline: 1.2400x; 1.0011x over previous
import jax
import jax.numpy as jnp
from jax.experimental import pallas as pl

BATCH = 4
SEQ_LEN = 8192
D_MODEL = 768
BS = 1024


def _add_body(x_ref, pos_ref, out_ref):
    out_ref[...] = x_ref[...] + pos_ref[...][None]


def kernel(x, pos_table):
    grid = (SEQ_LEN // BS,)
    return pl.pallas_call(
        _add_body,
        grid=grid,
        in_specs=[
            pl.BlockSpec((BATCH, BS, D_MODEL), lambda s: (0, s, 0)),
            pl.BlockSpec((BS, D_MODEL), lambda s: (s, 0)),
        ],
        out_specs=pl.BlockSpec((BATCH, BS, D_MODEL), lambda s: (0, s, 0)),
        out_shape=jax.ShapeDtypeStruct((BATCH, SEQ_LEN, D_MODEL), jnp.float32),
    )(x, pos_table)
